# Initial kernel scaffold; baseline (speedup 1.0000x reference)
import jax
import jax.numpy as jnp
from jax.experimental import pallas as pl


def kernel(x, Wg, bg, W1, b1, W2, b2, bias):
    raise NotImplementedError("write your pallas kernel here")


# speedup vs baseline: 3.1091x; 3.1091x over previous
import jax
import jax.numpy as jnp
from jax.experimental import pallas as pl
from jax.experimental.pallas import tpu as pltpu

B = 2048
D = 768
E = 16
H = 256


def _dense_body(x_ref, Wg_ref, bg_ref, bias_ref, W1_ref, b1_ref, W2_ref, b2_ref,
                logits_ref, idx_ref, out_ref, idx_s):
    e = pl.program_id(0)

    @pl.when(e == 0)
    def _():
        g = jnp.dot(x_ref[...], Wg_ref[...], preferred_element_type=jnp.float32)
        g = g + bg_ref[...]
        logits_ref[...] = g
        bsd = g + bias_ref[...]
        m = jnp.max(bsd, axis=1, keepdims=True)
        ids = jax.lax.broadcasted_iota(jnp.int32, (B, E), 1)
        cand = jnp.where(bsd == m, ids, E)
        am = jnp.min(cand, axis=1, keepdims=True)
        idx_ref[...] = am
        idx_s[...] = am

    h = jnp.dot(x_ref[...], W1_ref[0], preferred_element_type=jnp.float32)
    h = jnp.maximum(h + b1_ref[0], 0.0)
    o = jnp.dot(h, W2_ref[0], preferred_element_type=jnp.float32) + b2_ref[0]
    mask = idx_s[...] == e
    prev = jnp.where(e == 0, jnp.zeros_like(o), out_ref[...])
    out_ref[...] = jnp.where(mask, o, prev)


def kernel(x, Wg, bg, W1, b1, W2, b2, bias):
    logits, idx, out = pl.pallas_call(
        _dense_body,
        grid=(E,),
        in_specs=[
            pl.BlockSpec((B, D), lambda e: (0, 0)),
            pl.BlockSpec((D, E), lambda e: (0, 0)),
            pl.BlockSpec((1, E), lambda e: (0, 0)),
            pl.BlockSpec((1, E), lambda e: (0, 0)),
            pl.BlockSpec((1, D, H), lambda e: (e, 0, 0)),
            pl.BlockSpec((1, 1, H), lambda e: (e, 0, 0)),
            pl.BlockSpec((1, H, D), lambda e: (e, 0, 0)),
            pl.BlockSpec((1, 1, D), lambda e: (e, 0, 0)),
        ],
        out_specs=[
            pl.BlockSpec((B, E), lambda e: (0, 0)),
            pl.BlockSpec((B, 1), lambda e: (0, 0)),
            pl.BlockSpec((B, D), lambda e: (0, 0)),
        ],
        out_shape=[
            jax.ShapeDtypeStruct((B, E), jnp.float32),
            jax.ShapeDtypeStruct((B, 1), jnp.int32),
            jax.ShapeDtypeStruct((B, D), jnp.float32),
        ],
        scratch_shapes=[pltpu.VMEM((B, 1), jnp.int32)],
    )(x, Wg, bg.reshape(1, E), bias, W1, b1.reshape(E, 1, H), W2,
      b2.reshape(E, 1, D))
    return (out, logits, idx)
